# baseline (speedup 1.0000x reference)
import jax
import jax.numpy as jnp
from jax.experimental import pallas as pl
from jax.experimental.pallas import tpu as pltpu

_BR = 8


def _fill_body(x_ref, o_ref):
    o_ref[...] = x_ref[...]


def kernel(logits, attention_mask):
    B, V = logits.shape
    out = pl.pallas_call(
        _fill_body,
        grid=(B // _BR,),
        in_specs=[pl.BlockSpec((_BR, V), lambda i: (i, 0))],
        out_specs=pl.BlockSpec((_BR, V), lambda i: (i, 0)),
        out_shape=jax.ShapeDtypeStruct((B, V), jnp.float32),
        compiler_params=pltpu.CompilerParams(
            dimension_semantics=("parallel",),
        ),
    )(logits)
    return out
